# baseline (speedup 1.0000x reference)
import functools
import math

import jax
import jax.numpy as jnp
from jax import lax
from jax.experimental import pallas as pl
from jax.experimental.pallas import tpu as pltpu
from jax.experimental.pallas import tpu_sc as plsc

D_MODEL = 64
SCALE = math.sqrt(D_MODEL)
NUM_WORKERS = 32
SUB = 128
MACRO = 512
LANES = 16


def _scale_chunk(rows_v):
    rows_per_iter = 8

    def body(i, _):
        for rr in range(rows_per_iter):
            r = i * rows_per_iter + rr
            for cc in range(D_MODEL // LANES):
                sl = (r, pl.ds(cc * LANES, LANES))
                rows_v[sl] = rows_v[sl] * SCALE
        return 0

    lax.fori_loop(0, MACRO // rows_per_iter, body, 0)


def _make_sc_gather(batch):
    n_per_w = batch // NUM_WORKERS
    n_macro = n_per_w // MACRO
    subs_per_macro = MACRO // SUB
    idx_rows = n_per_w // SUB

    mesh = plsc.VectorSubcoreMesh(core_axis_name="c", subcore_axis_name="s")

    @functools.partial(
        pl.kernel,
        out_type=jax.ShapeDtypeStruct((batch, D_MODEL), jnp.float32),
        mesh=mesh,
        compiler_params=pltpu.CompilerParams(use_tc_tiling_on_sc=False),
        scratch_types=[
            pltpu.VMEM((idx_rows, SUB), jnp.int32),
            pltpu.VMEM((MACRO, D_MODEL), jnp.float32),
            pltpu.VMEM((MACRO, D_MODEL), jnp.float32),
            pltpu.SemaphoreType.DMA,
            pltpu.SemaphoreType.DMA,
            pltpu.SemaphoreType.DMA,
        ],
    )
    def gather_kernel(idx_hbm, table_hbm, out_hbm, idx_v, rows0, rows1,
                      isem, gsem, osem):
        wid = lax.axis_index("s") * 2 + lax.axis_index("c")
        base = wid * n_per_w

        pltpu.async_copy(
            idx_hbm.at[pl.ds(wid * idx_rows, idx_rows)], idx_v, isem
        ).wait()

        bufs = (rows0, rows1)

        def gather_chunk(g, buf):
            copies = []
            for j in range(subs_per_macro):
                copies.append(pltpu.async_copy(
                    table_hbm.at[idx_v.at[g * subs_per_macro + j]],
                    buf.at[pl.ds(j * SUB, SUB)],
                    gsem,
                ))
            for c in copies:
                c.wait()

        def start_out(g, buf):
            pltpu.async_copy(
                buf, out_hbm.at[pl.ds(base + g * MACRO, MACRO)], osem
            )

        def wait_out(g, buf):
            pltpu.make_async_copy(
                buf, out_hbm.at[pl.ds(base + g * MACRO, MACRO)], osem
            ).wait()

        for g in range(2):
            gather_chunk(g, bufs[g])
            _scale_chunk(bufs[g])
            start_out(g, bufs[g])

        def body(g2, _):
            for b in range(2):
                g = g2 * 2 + b
                buf = bufs[b]
                wait_out(g, buf)
                gather_chunk(g, buf)
                _scale_chunk(buf)
                start_out(g, buf)
            return 0

        lax.fori_loop(1, n_macro // 2, body, 0)

        for g in range(2):
            wait_out(g, bufs[g])

    return gather_kernel


def kernel(x, table):
    batch = x.size
    idx = x.reshape(batch // SUB, SUB).astype(jnp.int32)
    out = _make_sc_gather(batch)(idx, table)
    return out.reshape(*x.shape, D_MODEL)
